# Initial kernel scaffold; baseline (speedup 1.0000x reference)
import jax
import jax.numpy as jnp
from jax.experimental import pallas as pl


def kernel(node_features, edge_features, memory, source_nodes, timestamps, neighbors, edge_idxs, edge_times, time_w, time_b, Wq, Wk, Wv, fc1_w, fc1_b, fc2_w, fc2_b):
    raise NotImplementedError("write your pallas kernel here")


# speedup vs baseline: 8.5145x; 8.5145x over previous
import functools

import jax
import jax.numpy as jnp
from jax import lax
from jax.experimental import pallas as pl
from jax.experimental.pallas import tpu as pltpu
from jax.experimental.pallas import tpu_sc as plsc

_CHUNK = 128
_BB = 256

_COS_C = (0.9999999969535533, -19.73920657927625, 64.93920774107887,
          -85.45159123705025, 60.17829035794926, -26.003974317084214,
          6.575528195385702)


def _fast_cos(x):
    t = x * jnp.float32(1.0 / (2.0 * 3.141592653589793))
    t = t - jnp.round(t)
    u = t * t
    acc = jnp.full_like(u, jnp.float32(_COS_C[6]))
    for k in range(5, -1, -1):
        acc = acc * u + jnp.float32(_COS_C[k])
    return acc


def _add_body(a_ref, b_ref, o_ref):
    o_ref[...] = a_ref[...] + b_ref[...]


def _sc_gather(combined, edge_features, src_idx, ngh_idx, ef_idx):
    n_nodes, d = combined.shape
    n_edges, de = edge_features.shape
    b = src_idx.shape[0]
    r = ngh_idx.shape[0]
    info = plsc.get_sparse_core_info()
    nc, ns = info.num_cores, info.num_subcores
    nw = nc * ns
    src_per_w = b // nw
    ngh_per_w = r // nw
    n_src = src_per_w // _CHUNK
    n_ngh = ngh_per_w // _CHUNK
    mesh = plsc.VectorSubcoreMesh(core_axis_name="c", subcore_axis_name="s")

    @functools.partial(
        pl.kernel,
        out_type=(
            jax.ShapeDtypeStruct((b, d), jnp.float32),
            jax.ShapeDtypeStruct((r, d), jnp.float32),
            jax.ShapeDtypeStruct((r, de), jnp.float32),
        ),
        mesh=mesh,
        scratch_types=[
            pltpu.VMEM((ngh_per_w,), jnp.int32),
            pltpu.VMEM((ngh_per_w,), jnp.int32),
            pltpu.VMEM((src_per_w,), jnp.int32),
            pltpu.VMEM((_CHUNK, d), jnp.float32),
            pltpu.VMEM((_CHUNK, d), jnp.float32),
            pltpu.VMEM((_CHUNK, de), jnp.float32),
            pltpu.VMEM((_CHUNK, de), jnp.float32),
            pltpu.SemaphoreType.DMA,
            pltpu.SemaphoreType.DMA,
        ],
        compiler_params=pltpu.CompilerParams(use_tc_tiling_on_sc=False),
    )
    def k(comb_hbm, ef_hbm, sidx_hbm, nidx_hbm, eidx_hbm,
          src_out, ngh_out, ef_out,
          nidx_v, eidx_v, sidx_v, buf0, buf1, efb0, efb1, sem0, sem1):
        wid = lax.axis_index("s") * nc + lax.axis_index("c")
        nbase = wid * ngh_per_w
        sbase = wid * src_per_w
        pltpu.sync_copy(nidx_hbm.at[pl.ds(nbase, ngh_per_w)], nidx_v)
        pltpu.sync_copy(eidx_hbm.at[pl.ds(nbase, ngh_per_w)], eidx_v)
        pltpu.sync_copy(sidx_hbm.at[pl.ds(sbase, src_per_w)], sidx_v)

        def ngh_pair(j, c):
            o0 = j * (2 * _CHUNK)
            o1 = o0 + _CHUNK
            cp0 = pltpu.async_copy(
                comb_hbm.at[nidx_v.at[pl.ds(o0, _CHUNK)]], buf0, sem0)
            cp1 = pltpu.async_copy(
                comb_hbm.at[nidx_v.at[pl.ds(o1, _CHUNK)]], buf1, sem1)
            cp0.wait()
            pltpu.sync_copy(buf0, ngh_out.at[pl.ds(nbase + o0, _CHUNK)])
            cp1.wait()
            pltpu.sync_copy(buf1, ngh_out.at[pl.ds(nbase + o1, _CHUNK)])
            return c
        lax.fori_loop(0, n_ngh // 2, ngh_pair, 0, unroll=False)

        def ef_pair(j, c):
            o0 = j * (2 * _CHUNK)
            o1 = o0 + _CHUNK
            cp0 = pltpu.async_copy(
                ef_hbm.at[eidx_v.at[pl.ds(o0, _CHUNK)]], efb0, sem0)
            cp1 = pltpu.async_copy(
                ef_hbm.at[eidx_v.at[pl.ds(o1, _CHUNK)]], efb1, sem1)
            cp0.wait()
            pltpu.sync_copy(efb0, ef_out.at[pl.ds(nbase + o0, _CHUNK)])
            cp1.wait()
            pltpu.sync_copy(efb1, ef_out.at[pl.ds(nbase + o1, _CHUNK)])
            return c
        lax.fori_loop(0, n_ngh // 2, ef_pair, 0, unroll=False)

        def src_chunk(j, c):
            o0 = j * _CHUNK
            cp0 = pltpu.async_copy(
                comb_hbm.at[sidx_v.at[pl.ds(o0, _CHUNK)]], buf0, sem0)
            cp0.wait()
            pltpu.sync_copy(buf0, src_out.at[pl.ds(sbase + o0, _CHUNK)])
            return c
        lax.fori_loop(0, n_src, src_chunk, 0, unroll=False)

    return k(combined, edge_features, src_idx, ngh_idx, ef_idx)


def _attn_body(ts_ref, et_ref, eidx_ref, src_ref, ngh_ref, ef_ref,
               tw_ref, tb_ref, wq_ref, wkv_ref, fc1_ref, f1b_ref,
               fc2_ref, f2b_ref, o_ref):
    f32 = jnp.float32
    ngh_n, bb, d = ngh_ref.shape
    de = ef_ref.shape[2]
    dt = tw_ref.shape[1]
    dh = d // 2
    src = src_ref[...]
    tw = tw_ref[...]
    tb = tb_ref[...]
    qb = jnp.dot(jnp.cos(tb), wq_ref[d:, :], preferred_element_type=f32)
    q = jnp.dot(src, wq_ref[:d, :], preferred_element_type=f32) + qb
    delta = ts_ref[...] - et_ref[...]
    mask = jnp.where(eidx_ref[...] == 0, -1e10, 0.0).astype(f32)
    q0 = q[:, :dh]
    q1 = q[:, dh:]
    scale = f32(1.0) / jnp.sqrt(f32(dh))
    vs = []
    s0l = []
    s1l = []
    for n in range(ngh_n):
        ete = _fast_cos(delta[:, n:n + 1] * tw + tb)
        kv = (jnp.dot(ngh_ref[n], wkv_ref[:d, :], preferred_element_type=f32)
              + jnp.dot(ete, wkv_ref[d:d + dt, :], preferred_element_type=f32)
              + jnp.dot(ef_ref[n], wkv_ref[d + dt:, :],
                        preferred_element_type=f32))
        kk = kv[:, :d]
        vs.append(kv[:, d:])
        s0l.append(jnp.sum(q0 * kk[:, :dh], axis=1, keepdims=True))
        s1l.append(jnp.sum(q1 * kk[:, dh:], axis=1, keepdims=True))
    s0 = jnp.concatenate(s0l, axis=1) * scale + mask
    s1 = jnp.concatenate(s1l, axis=1) * scale + mask

    def _softmax(s):
        m = jnp.max(s, axis=1, keepdims=True)
        e = jnp.exp(s - m)
        return e / jnp.sum(e, axis=1, keepdims=True)

    a0 = _softmax(s0)
    a1 = _softmax(s1)
    acc = jnp.zeros((bb, d), dtype=f32)
    for n in range(ngh_n):
        acc = acc + jnp.concatenate(
            [a0[:, n:n + 1] * vs[n][:, :dh], a1[:, n:n + 1] * vs[n][:, dh:]],
            axis=1)
    h = (jnp.dot(acc, fc1_ref[:d, :], preferred_element_type=f32)
         + jnp.dot(src, fc1_ref[d:, :], preferred_element_type=f32)
         + f1b_ref[...])
    h = jnp.maximum(h, f32(0.0))
    o_ref[...] = jnp.dot(h, fc2_ref[...], preferred_element_type=f32) + f2b_ref[...]


def _attn_specs(b, ngh, d, de, dt):
    bb = _BB
    grid = (b // bb,)
    full = lambda i: (0, 0)
    in_specs = [
        pl.BlockSpec((bb, 1), lambda i: (i, 0)),
        pl.BlockSpec((bb, ngh), lambda i: (i, 0)),
        pl.BlockSpec((bb, ngh), lambda i: (i, 0)),
        pl.BlockSpec((bb, d), lambda i: (i, 0)),
        pl.BlockSpec((ngh, bb, d), lambda i: (0, i, 0)),
        pl.BlockSpec((ngh, bb, de), lambda i: (0, i, 0)),
        pl.BlockSpec((1, dt), full),
        pl.BlockSpec((1, dt), full),
        pl.BlockSpec((d + dt, d), full),
        pl.BlockSpec((d + dt + de, 2 * d), full),
        pl.BlockSpec((2 * d, d), full),
        pl.BlockSpec((1, d), full),
        pl.BlockSpec((d, d), full),
        pl.BlockSpec((1, d), full),
    ]
    out_spec = pl.BlockSpec((bb, d), lambda i: (i, 0))
    return grid, in_specs, out_spec


def kernel(node_features, edge_features, memory, source_nodes, timestamps,
           neighbors, edge_idxs, edge_times, time_w, time_b, Wq, Wk, Wv,
           fc1_w, fc1_b, fc2_w, fc2_b):
    n_nodes, d = node_features.shape
    n_edges, de = edge_features.shape
    b, ngh = neighbors.shape
    dt = time_w.shape[0]

    combined = pl.pallas_call(
        _add_body,
        out_shape=jax.ShapeDtypeStruct((n_nodes, d), jnp.float32),
    )(node_features, memory)

    src_idx = source_nodes.astype(jnp.int32)
    ngh_idx = neighbors.T.reshape(-1).astype(jnp.int32)
    ef_idx = edge_idxs.T.reshape(-1).astype(jnp.int32)
    src_rows, ngh_rows, ef_rows = _sc_gather(
        combined, edge_features, src_idx, ngh_idx, ef_idx)
    ngh3 = ngh_rows.reshape(ngh, b, d)
    ef3 = ef_rows.reshape(ngh, b, de)

    grid, in_specs, out_spec = _attn_specs(b, ngh, d, de, dt)
    wkv = jnp.concatenate([Wk, Wv], axis=1)
    out = pl.pallas_call(
        _attn_body,
        grid=grid,
        in_specs=in_specs,
        out_specs=out_spec,
        out_shape=jax.ShapeDtypeStruct((b, d), jnp.float32),
    )(timestamps[:, None], edge_times, edge_idxs, src_rows, ngh3, ef3,
      time_w[None, :], time_b[None, :], Wq, wkv, fc1_w, fc1_b[None, :],
      fc2_w, fc2_b[None, :])
    return out
